# Initial kernel scaffold; baseline (speedup 1.0000x reference)
import jax
import jax.numpy as jnp
from jax.experimental import pallas as pl


def kernel(dense_input, sparse_input, embed_tables, W_lin, b_lin, W0, b0, W1, b1, W2, b2, W_out):
    raise NotImplementedError("write your pallas kernel here")


# speedup vs baseline: 1.0237x; 1.0237x over previous
import functools

import jax
import jax.numpy as jnp
from jax import lax
from jax.experimental import pallas as pl
from jax.experimental.pallas import tpu as pltpu
from jax.experimental.pallas import tpu_sc as plsc

_N_DENSE = 13


def _make_sc_gather(total_rows, d):
    info = plsc.get_sparse_core_info()
    nc, ns = info.num_cores, info.num_subcores
    nw = nc * ns
    rows_per_w = total_rows // nw
    mesh = plsc.VectorSubcoreMesh(core_axis_name="c", subcore_axis_name="s")

    @functools.partial(
        pl.kernel,
        mesh=mesh,
        compiler_params=pltpu.CompilerParams(use_tc_tiling_on_sc=False),
        out_type=jax.ShapeDtypeStruct((total_rows, d), jnp.float32),
        scratch_types=[
            pltpu.VMEM((rows_per_w,), jnp.int32),
            pltpu.VMEM((rows_per_w, d), jnp.float32),
            pltpu.SemaphoreType.DMA,
        ],
    )
    def gather_kernel(table_hbm, idx_hbm, out_hbm, idx_v, rows_v, sem):
        wid = lax.axis_index("s") * nc + lax.axis_index("c")
        base = wid * rows_per_w
        pltpu.sync_copy(idx_hbm.at[pl.ds(base, rows_per_w)], idx_v)
        pltpu.async_copy(table_hbm.at[idx_v], rows_v, sem).wait()
        pltpu.sync_copy(rows_v, out_hbm.at[pl.ds(base, rows_per_w)])

    return gather_kernel


def _tc_body(dense_ref, emb_ref, wlin_ref, blin_ref, w0d_ref, w0s_ref,
             b0_ref, w1_ref, b1_ref, w2_ref, b2_ref, wout_ref, out_ref):
    emb = emb_ref[...]
    d = dense_ref[...]
    s = jnp.sum(emb, axis=1, keepdims=True)
    sq = jnp.sum(emb * emb, axis=1, keepdims=True)
    lin = jnp.dot(d, wlin_ref[...], preferred_element_type=jnp.float32) + blin_ref[...]
    h = jnp.dot(d, w0d_ref[...], preferred_element_type=jnp.float32)
    h += jnp.dot(emb, w0s_ref[...], preferred_element_type=jnp.float32)
    h = jnp.maximum(h + b0_ref[...], 0.0)
    h = jnp.maximum(
        jnp.dot(h, w1_ref[...], preferred_element_type=jnp.float32) + b1_ref[...], 0.0)
    h = jnp.maximum(
        jnp.dot(h, w2_ref[...], preferred_element_type=jnp.float32) + b2_ref[...], 0.0)
    dnn = jnp.dot(h, wout_ref[...], preferred_element_type=jnp.float32)
    z = lin + s + 0.5 * (s * s - sq) + dnn
    out_ref[...] = jax.nn.sigmoid(z)


def _tc_forward(dense_input, emb, W_lin, b_lin, W0d, W0s, b0, W1, b1, W2, b2, W_out,
                block_b=512):
    b = dense_input.shape[0]
    n_dense = dense_input.shape[1]
    fe = emb.shape[1]
    u0, u1, u2 = W0s.shape[1], W1.shape[1], W2.shape[1]
    grid = (b // block_b,)
    full = lambda shape: pl.BlockSpec(shape, lambda i: (0, 0))
    return pl.pallas_call(
        _tc_body,
        grid=grid,
        in_specs=[
            pl.BlockSpec((block_b, n_dense), lambda i: (i, 0)),
            pl.BlockSpec((block_b, fe), lambda i: (i, 0)),
            full((n_dense, 1)),
            full((1, 1)),
            full((n_dense, u0)),
            full((fe, u0)),
            full((1, u0)),
            full((u0, u1)),
            full((1, u1)),
            full((u1, u2)),
            full((1, u2)),
            full((u2, 1)),
        ],
        out_specs=pl.BlockSpec((block_b, 1), lambda i: (i, 0)),
        out_shape=jax.ShapeDtypeStruct((b, 1), jnp.float32),
    )(dense_input, emb, W_lin, b_lin.reshape(1, 1), W0d, W0s,
      b0.reshape(1, u0), W1, b1.reshape(1, u1), W2, b2.reshape(1, u2), W_out)


def kernel(dense_input, sparse_input, embed_tables, W_lin, b_lin,
           W0, b0, W1, b1, W2, b2, W_out):
    b, f = sparse_input.shape
    v, d = embed_tables.shape[1], embed_tables.shape[2]
    flat_idx = (sparse_input + jnp.arange(f, dtype=jnp.int32)[None, :] * v).reshape(-1)
    table = embed_tables.reshape(f * v, d)
    gathered = _make_sc_gather(b * f, d)(table, flat_idx)
    emb = gathered.reshape(b, f * d)
    W0d = W0[:_N_DENSE]
    W0s = W0[_N_DENSE:]
    return _tc_forward(dense_input, emb, W_lin, b_lin, W0d, W0s, b0, W1, b1, W2, b2, W_out)
